# baseline (speedup 1.0000x reference)
import functools

import jax
import jax.numpy as jnp
from jax import lax
from jax.experimental import pallas as pl
from jax.experimental.pallas import tpu as pltpu
from jax.experimental.pallas import tpu_sc as plsc

N = 10000
E = 320000
D = 128

NC = 2
NS = 16
CHUNK = 128
NCHUNK = 80
E_TILE = CHUNK * NCHUNK
E_SC = E_TILE * NS
E_PAD = E_SC * NC
N_PAD = 10240
ZROWS = N_PAD // NS
CW = 128

_mesh = plsc.VectorSubcoreMesh(core_axis_name="c", subcore_axis_name="s")


NBUF = 2
DBUF = 4


@functools.partial(
    pl.kernel,
    mesh=_mesh,
    out_type=jax.ShapeDtypeStruct((NC, N_PAD, D), jnp.float32),
    scratch_types=[
        pltpu.VMEM((CHUNK,), jnp.int32),
        pltpu.VMEM((CHUNK,), jnp.int32),
        pltpu.VMEM((CHUNK, D), jnp.float32),
        pltpu.VMEM_SHARED((N_PAD, D), jnp.float32),
        pltpu.SemaphoreType.DMA,
    ],
)
def _segsum(g_hbm, src_hbm, dst_hbm, out_hbm, src_v, dst_v, rows_v,
            acc_sh, sem):
    cid = lax.axis_index("c")
    sid = lax.axis_index("s")

    def _zr(i, _):
        def _zc(j, _):
            rows_v[i, pl.ds(j * 16, 16)] = jnp.zeros((16,), jnp.float32)
            return 0
        return lax.fori_loop(0, D // 16, _zc, 0)
    lax.fori_loop(0, CHUNK, _zr, 0)

    def _zcopy(k, _):
        row0 = sid * ZROWS + k * CHUNK
        pltpu.sync_copy(rows_v, acc_sh.at[pl.ds(row0, CHUNK)])
        return 0
    lax.fori_loop(0, ZROWS // CHUNK, _zcopy, 0)
    plsc.subcore_barrier()

    base = (cid * NS + sid) * E_TILE

    def _chunk(c, _):
        off = pl.multiple_of(base + c * CHUNK, CHUNK)
        pltpu.sync_copy(src_hbm.at[pl.ds(off, CHUNK)], src_v)
        pltpu.sync_copy(dst_hbm.at[pl.ds(off, CHUNK)], dst_v)
        pltpu.async_copy(g_hbm.at[src_v], rows_v, sem).wait()
        pltpu.sync_copy(rows_v, acc_sh.at[dst_v], add=True)
        return 0
    lax.fori_loop(0, NCHUNK, _chunk, 0)
    plsc.subcore_barrier()

    row0 = sid * ZROWS
    pltpu.sync_copy(acc_sh.at[pl.ds(row0, ZROWS)],
                    out_hbm.at[cid, pl.ds(row0, ZROWS)])


@functools.partial(
    pl.kernel,
    mesh=_mesh,
    out_type=jax.ShapeDtypeStruct((NC, N_PAD, CW), jnp.float32),
    scratch_types=[
        pltpu.VMEM((NCHUNK, CHUNK), jnp.int32),
        pltpu.VMEM((CHUNK, CW), jnp.float32),
        pltpu.VMEM((16, CW), jnp.float32),
        pltpu.VMEM_SHARED((N_PAD, CW), jnp.float32),
        pltpu.SemaphoreType.DMA,
    ],
)
def _segcnt(dst_hbm, out_hbm, dst_v, ones_v, zb_v, cnt_sh, isem):
    cid = lax.axis_index("c")
    sid = lax.axis_index("s")

    tbase = (cid * NS + sid) * NCHUNK
    icp = pltpu.async_copy(dst_hbm.at[pl.ds(tbase, NCHUNK)], dst_v, isem)

    def _fill(i, _):
        def _fc(j, _):
            zb_v[i, pl.ds(j * 16, 16)] = jnp.zeros((16,), jnp.float32)
            return 0
        return lax.fori_loop(0, CW // 16, _fc, 0)
    lax.fori_loop(0, 16, _fill, 0)

    def _fill1(i, _):
        def _fc(j, _):
            ones_v[i, pl.ds(j * 16, 16)] = jnp.ones((16,), jnp.float32)
            return 0
        return lax.fori_loop(0, CW // 16, _fc, 0)
    lax.fori_loop(0, CHUNK, _fill1, 0)

    def _zcopy(k, _):
        row0 = sid * ZROWS + k * 16
        pltpu.sync_copy(zb_v, cnt_sh.at[pl.ds(row0, 16)])
        return 0
    lax.fori_loop(0, ZROWS // 16, _zcopy, 0)
    icp.wait()
    plsc.subcore_barrier()

    def _chunk(c, _):
        pltpu.sync_copy(ones_v, cnt_sh.at[dst_v.at[c]], add=True)
        return 0
    lax.fori_loop(0, NCHUNK, _chunk, 0)
    plsc.subcore_barrier()

    row0 = sid * ZROWS
    pltpu.sync_copy(cnt_sh.at[pl.ds(row0, ZROWS)],
                    out_hbm.at[cid, pl.ds(row0, ZROWS)])


BM = 2000


def _mm_body(h_ref, w_ref, b_ref, g_ref, r_ref):
    res = jnp.dot(h_ref[...], w_ref[...], preferred_element_type=jnp.float32)
    g_ref[...] = res[:, :D]
    r_ref[...] = res[:, D:] + b_ref[...]


def _mm(h, wcat, b):
    return pl.pallas_call(
        _mm_body,
        grid=(N // BM,),
        in_specs=[
            pl.BlockSpec((BM, D), lambda i: (i, 0)),
            pl.BlockSpec((D, 2 * D), lambda i: (0, 0)),
            pl.BlockSpec((1, D), lambda i: (0, 0)),
        ],
        out_specs=[
            pl.BlockSpec((BM, D), lambda i: (i, 0)),
            pl.BlockSpec((BM, D), lambda i: (i, 0)),
        ],
        out_shape=[
            jax.ShapeDtypeStruct((N, D), jnp.float32),
            jax.ShapeDtypeStruct((N, D), jnp.float32),
        ],
    )(h, wcat, b)


def _combine_body(p_ref, c_ref, r_ref, o_ref, *, relu):
    s = p_ref[0] + p_ref[1]
    cnt = c_ref[0, :, 0:1] + c_ref[1, :, 0:1]
    inv = 1.0 / jnp.maximum(cnt, 1.0)
    o = s * inv + r_ref[...]
    if relu:
        o = jnp.maximum(o, 0.0)
    o_ref[...] = o


def _combine(p, cnt, r, relu):
    return pl.pallas_call(
        functools.partial(_combine_body, relu=relu),
        grid=(N // BM,),
        in_specs=[
            pl.BlockSpec((NC, BM, D), lambda i: (0, i, 0)),
            pl.BlockSpec((NC, BM, CW), lambda i: (0, i, 0)),
            pl.BlockSpec((BM, D), lambda i: (i, 0)),
        ],
        out_specs=pl.BlockSpec((BM, D), lambda i: (i, 0)),
        out_shape=jax.ShapeDtypeStruct((N, D), jnp.float32),
    )(p, cnt, r)


def kernel(x, edge_index, W_l0, b_l0, W_r0, W_l1, b_l1, W_r1, W_l2, b_l2,
           W_r2):
    src = edge_index[0]
    dst = edge_index[1]
    pad = E_PAD - E
    src_p = jnp.concatenate([src, jnp.zeros((pad,), jnp.int32)])
    dst_p = jnp.concatenate([dst, jnp.full((pad,), N, jnp.int32)])
    src2 = src_p.reshape(E_PAD // CHUNK, CHUNK)
    dst2 = dst_p.reshape(E_PAD // CHUNK, CHUNK)

    cnt = _segcnt(dst2)

    h = x
    for (wl, bl, wr, relu) in (
        (W_l0, b_l0, W_r0, True),
        (W_l1, b_l1, W_r1, True),
        (W_l2, b_l2, W_r2, False),
    ):
        wcat = jnp.concatenate([wl.T, wr.T], axis=1)
        g, r = _mm(h, wcat, bl.reshape(1, D))
        p = _segsum(g, src_p, dst_p)
        h = _combine(p, cnt, r, relu)
    return h


# speedup vs baseline: 1.4658x; 1.4658x over previous
import functools

import jax
import jax.numpy as jnp
from jax import lax
from jax.experimental import pallas as pl
from jax.experimental.pallas import tpu as pltpu
from jax.experimental.pallas import tpu_sc as plsc

N = 10000
E = 320000
D = 128

NC = 2
NS = 16
CHUNK = 128
NCHUNK = 79
E_TILE = CHUNK * NCHUNK
E_SC = E_TILE * NS
E_PAD = E_SC * NC
N_PAD = 10240
ZROWS = N_PAD // NS
CW = 128

_mesh = plsc.VectorSubcoreMesh(core_axis_name="c", subcore_axis_name="s")


@functools.partial(
    pl.kernel,
    mesh=_mesh,
    out_type=jax.ShapeDtypeStruct((NC, N_PAD, D), jnp.float32),
    scratch_types=[
        pltpu.VMEM((CHUNK,), jnp.int32),
        pltpu.VMEM((CHUNK,), jnp.int32),
        pltpu.VMEM((CHUNK, D), jnp.float32),
        pltpu.VMEM((16, D), jnp.float32),
        pltpu.VMEM_SHARED((N_PAD, D), jnp.float32),
        pltpu.SemaphoreType.DMA,
    ],
)
def _segsum(g_hbm, src_hbm, dst_hbm, out_hbm, src_v, dst_v, rows_v, zb_v,
            acc_sh, sem):
    cid = lax.axis_index("c")
    sid = lax.axis_index("s")

    def _zr(i, _):
        def _zc(j, _):
            zb_v[i, pl.ds(j * 16, 16)] = jnp.zeros((16,), jnp.float32)
            return 0
        return lax.fori_loop(0, D // 16, _zc, 0)
    lax.fori_loop(0, 16, _zr, 0)

    def _zcopy(k, _):
        row0 = sid * ZROWS + k * 16
        pltpu.sync_copy(zb_v, acc_sh.at[pl.ds(row0, 16)])
        return 0
    lax.fori_loop(0, ZROWS // 16, _zcopy, 0)
    plsc.subcore_barrier()

    base = cid * E_SC + sid * E_TILE

    def _chunk(c, _):
        off = pl.multiple_of(base + c * CHUNK, CHUNK)
        pltpu.sync_copy(src_hbm.at[pl.ds(off, CHUNK)], src_v)
        pltpu.sync_copy(dst_hbm.at[pl.ds(off, CHUNK)], dst_v)
        pltpu.async_copy(g_hbm.at[src_v], rows_v, sem).wait()
        pltpu.sync_copy(rows_v, acc_sh.at[dst_v], add=True)
        return 0
    lax.fori_loop(0, NCHUNK, _chunk, 0)
    plsc.subcore_barrier()

    row0 = sid * ZROWS
    pltpu.sync_copy(acc_sh.at[pl.ds(row0, ZROWS)],
                    out_hbm.at[cid, pl.ds(row0, ZROWS)])


@functools.partial(
    pl.kernel,
    mesh=_mesh,
    out_type=jax.ShapeDtypeStruct((NC, N_PAD, CW), jnp.float32),
    scratch_types=[
        pltpu.VMEM((CHUNK,), jnp.int32),
        pltpu.VMEM((CHUNK, CW), jnp.float32),
        pltpu.VMEM((16, CW), jnp.float32),
        pltpu.VMEM_SHARED((N_PAD, CW), jnp.float32),
    ],
)
def _segcnt(dst_hbm, out_hbm, dst_v, ones_v, zb_v, cnt_sh):
    cid = lax.axis_index("c")
    sid = lax.axis_index("s")

    def _fill(i, _):
        def _fc(j, _):
            zb_v[i, pl.ds(j * 16, 16)] = jnp.zeros((16,), jnp.float32)
            return 0
        return lax.fori_loop(0, CW // 16, _fc, 0)
    lax.fori_loop(0, 16, _fill, 0)

    def _fill1(i, _):
        def _fc(j, _):
            ones_v[i, pl.ds(j * 16, 16)] = jnp.ones((16,), jnp.float32)
            return 0
        return lax.fori_loop(0, CW // 16, _fc, 0)
    lax.fori_loop(0, CHUNK, _fill1, 0)

    def _zcopy(k, _):
        row0 = sid * ZROWS + k * 16
        pltpu.sync_copy(zb_v, cnt_sh.at[pl.ds(row0, 16)])
        return 0
    lax.fori_loop(0, ZROWS // 16, _zcopy, 0)
    plsc.subcore_barrier()

    base = cid * E_SC + sid * E_TILE

    def _chunk(c, _):
        off = pl.multiple_of(base + c * CHUNK, CHUNK)
        pltpu.sync_copy(dst_hbm.at[pl.ds(off, CHUNK)], dst_v)
        pltpu.sync_copy(ones_v, cnt_sh.at[dst_v], add=True)
        return 0
    lax.fori_loop(0, NCHUNK, _chunk, 0)
    plsc.subcore_barrier()

    row0 = sid * ZROWS
    pltpu.sync_copy(cnt_sh.at[pl.ds(row0, ZROWS)],
                    out_hbm.at[cid, pl.ds(row0, ZROWS)])


BM = 2000


def _mm_body(h_ref, w_ref, b_ref, g_ref, r_ref):
    res = jnp.dot(h_ref[...], w_ref[...], preferred_element_type=jnp.float32)
    g_ref[...] = res[:, :D]
    r_ref[...] = res[:, D:] + b_ref[...]


def _mm(h, wcat, b):
    return pl.pallas_call(
        _mm_body,
        grid=(N // BM,),
        in_specs=[
            pl.BlockSpec((BM, D), lambda i: (i, 0)),
            pl.BlockSpec((D, 2 * D), lambda i: (0, 0)),
            pl.BlockSpec((1, D), lambda i: (0, 0)),
        ],
        out_specs=[
            pl.BlockSpec((BM, D), lambda i: (i, 0)),
            pl.BlockSpec((BM, D), lambda i: (i, 0)),
        ],
        out_shape=[
            jax.ShapeDtypeStruct((N, D), jnp.float32),
            jax.ShapeDtypeStruct((N, D), jnp.float32),
        ],
    )(h, wcat, b)


def _combine_body(p_ref, c_ref, r_ref, o_ref, *, relu):
    s = p_ref[0] + p_ref[1]
    cnt = c_ref[0, :, 0:1] + c_ref[1, :, 0:1]
    inv = 1.0 / jnp.maximum(cnt, 1.0)
    o = s * inv + r_ref[...]
    if relu:
        o = jnp.maximum(o, 0.0)
    o_ref[...] = o


def _combine(p, cnt, r, relu):
    return pl.pallas_call(
        functools.partial(_combine_body, relu=relu),
        grid=(N // BM,),
        in_specs=[
            pl.BlockSpec((NC, BM, D), lambda i: (0, i, 0)),
            pl.BlockSpec((NC, BM, CW), lambda i: (0, i, 0)),
            pl.BlockSpec((BM, D), lambda i: (i, 0)),
        ],
        out_specs=pl.BlockSpec((BM, D), lambda i: (i, 0)),
        out_shape=jax.ShapeDtypeStruct((N, D), jnp.float32),
    )(p, cnt, r)


def kernel(x, edge_index, W_l0, b_l0, W_r0, W_l1, b_l1, W_r1, W_l2, b_l2,
           W_r2):
    src = edge_index[0]
    dst = edge_index[1]
    pad = E_PAD - E
    src_p = jnp.concatenate([src, jnp.zeros((pad,), jnp.int32)])
    dst_p = jnp.concatenate([dst, jnp.full((pad,), N, jnp.int32)])

    cnt = _segcnt(dst_p)

    h = x
    for (wl, bl, wr, relu) in (
        (W_l0, b_l0, W_r0, True),
        (W_l1, b_l1, W_r1, True),
        (W_l2, b_l2, W_r2, False),
    ):
        wcat = jnp.concatenate([wl.T, wr.T], axis=1)
        g, r = _mm(h, wcat, bl.reshape(1, D))
        p = _segsum(g, src_p, dst_p)
        h = _combine(p, cnt, r, relu)
    return h
